# baseline (speedup 1.0000x reference)
import functools

import jax
import jax.numpy as jnp
from jax import lax
from jax.experimental import pallas as pl
from jax.experimental.pallas import tpu as pltpu
from jax.experimental.pallas import tpu_sc as plsc

BRUN = 128
SGRP = 8
CHUNK = 128
NBUF = 10


def _emb_transposed(wordsT, table, seq, batch, dim):
    info = plsc.get_sparse_core_info()
    num_cores = info.num_cores
    n_t = seq // SGRP
    n_tc = batch // BRUN
    mesh = plsc.VectorSubcoreMesh(core_axis_name="c", subcore_axis_name="s")

    @functools.partial(
        pl.kernel,
        out_type=jax.ShapeDtypeStruct((seq, dim // 8, n_tc, 8, BRUN), jnp.float32),
        mesh=mesh,
        scratch_types=[
            pltpu.VMEM((2, SGRP, BRUN), jnp.int32),
            pltpu.VMEM((SGRP, BRUN, dim), jnp.float32),
            pltpu.VMEM((2, dim // 8, 8, BRUN), jnp.float32),
            pltpu.SemaphoreType.DMA((2,)),
            pltpu.SemaphoreType.DMA((SGRP,)),
            pltpu.SemaphoreType.DMA((2,)),
        ],
        compiler_params=pltpu.CompilerParams(
            use_tc_tiling_on_sc=False, needs_layout_passes=False
        ),
    )
    def emb(wordsT_hbm, table_hbm, out_hbm, ib, gb, tb, isem, gsem, wsem):
        w = lax.axis_index("s") * num_cores + lax.axis_index("c")
        b0 = w * BRUN
        iota16 = lax.iota(jnp.int32, 16)
        rows_lb = [lb * 16 + iota16 for lb in range(8)]

        def ib_copy(t, j):
            return pltpu.make_async_copy(
                wordsT_hbm.at[pl.ds(SGRP * t, SGRP), pl.ds(b0, BRUN)],
                ib.at[j],
                isem.at[j],
            )

        def gather(j_ib, ss):
            return pltpu.make_async_copy(
                table_hbm.at[ib.at[j_ib, ss]], gb.at[ss], gsem.at[ss]
            )

        def write(s, kt):
            return pltpu.make_async_copy(
                tb.at[kt], out_hbm.at[s, :, w], wsem.at[kt]
            )

        def transpose(ss, kt):
            kv = jnp.full((16,), ss, jnp.int32)

            def body(c, _):
                col = jnp.full((16,), c, jnp.int32)
                tr = lax.div(c, 8)
                sub = lax.rem(c, 8)
                vs = [
                    plsc.load_gather(gb, [kv, rows_lb[lb], col])
                    for lb in range(8)
                ]
                for lb in range(8):
                    tb[kt, tr, sub, pl.ds(lb * 16, 16)] = vs[lb]
                return _

            lax.fori_loop(0, dim, body, None)

        ib_copy(0, 0).start()
        ib_copy(0, 0).wait()
        for ss in range(SGRP):
            gather(0, ss).start()

        def t_body(t, _):
            j = lax.rem(t, 2)

            @pl.when(t + 1 < n_t)
            def _():
                ib_copy(t + 1, 1 - j).start()

            for ss in range(SGRP):
                s = SGRP * t + ss
                kt = ss % 2
                gather(j, ss).wait()

                @pl.when(s >= 2)
                def _():
                    write(s - 2, kt).wait()

                transpose(ss, kt)
                write(s, kt).start()

                if ss == 0:

                    @pl.when(t + 1 < n_t)
                    def _():
                        ib_copy(t + 1, 1 - j).wait()

                @pl.when(t + 1 < n_t)
                def _():
                    gather(1 - j, ss).start()

            return _

        lax.fori_loop(0, n_t, t_body, None)
        write(seq - 2, 0).wait()
        write(seq - 1, 1).wait()

    return emb(wordsT, table)


def _emb_rowgather(idx3, weights, n_per_w, n_chunks, dim):
    info = plsc.get_sparse_core_info()
    num_cores = info.num_cores
    num_workers = info.num_cores * info.num_subcores
    n_total = num_workers * n_per_w
    mesh = plsc.VectorSubcoreMesh(core_axis_name="c", subcore_axis_name="s")

    @functools.partial(
        pl.kernel,
        out_type=jax.ShapeDtypeStruct((n_total, dim), jnp.float32),
        mesh=mesh,
        scratch_types=[
            pltpu.VMEM((n_chunks, CHUNK), jnp.int32),
            pltpu.VMEM((NBUF, CHUNK, dim), jnp.float32),
            pltpu.SemaphoreType.DMA((NBUF,)),
        ],
        compiler_params=pltpu.CompilerParams(use_tc_tiling_on_sc=False),
    )
    def emb(idx_hbm, table_hbm, out_hbm, idx_v, rows_v, gsem):
        wid = lax.axis_index("s") * num_cores + lax.axis_index("c")
        base = wid * n_per_w
        pltpu.sync_copy(idx_hbm.at[wid], idx_v)

        def gather(c, b):
            return pltpu.make_async_copy(
                table_hbm.at[idx_v.at[c]], rows_v.at[b], gsem.at[b]
            )

        for b in range(min(NBUF, n_chunks)):
            gather(b, b).start()

        def group(g, _):
            for b in range(NBUF):
                c = g * NBUF + b
                gather(c, b).wait()
                pltpu.sync_copy(
                    rows_v.at[b], out_hbm.at[pl.ds(base + c * CHUNK, CHUNK)]
                )
                nxt = c + NBUF

                @pl.when(nxt < n_chunks)
                def _():
                    gather(nxt, b).start()

            return _

        n_full = n_chunks // NBUF
        lax.fori_loop(0, n_full, group, None)
        for b in range(n_chunks % NBUF):
            c = n_full * NBUF + b
            gather(c, b).wait()
            pltpu.sync_copy(
                rows_v.at[b], out_hbm.at[pl.ds(base + c * CHUNK, CHUNK)]
            )

    return emb(idx3, weights)


def kernel(words, weights):
    batch, seq = words.shape
    _, dim = weights.shape
    info = plsc.get_sparse_core_info()
    num_workers = info.num_cores * info.num_subcores

    if (
        dim == 64
        and batch == num_workers * BRUN
        and seq % SGRP == 0
        and seq >= 2 * SGRP
    ):
        wordsT = words.T.astype(jnp.int32)
        out5 = _emb_transposed(wordsT, weights, seq, batch, dim)
        out = out5.transpose(2, 4, 0, 1, 3).reshape(batch, seq, dim)
        return out

    n = batch * seq
    flat = words.reshape(n).astype(jnp.int32)
    tile = num_workers * CHUNK
    n_pad = ((n + tile - 1) // tile) * tile
    if n_pad != n:
        flat = jnp.concatenate([flat, jnp.zeros(n_pad - n, jnp.int32)])
    n_per_w = n_pad // num_workers
    n_chunks = n_per_w // CHUNK

    idx3 = flat.reshape(num_workers, n_chunks, CHUNK)
    out = _emb_rowgather(idx3, weights, n_per_w, n_chunks, dim)
    return out[:n].reshape(batch, seq, dim)


# speedup vs baseline: 1.7963x; 1.7963x over previous
import functools

import jax
import jax.numpy as jnp
from jax import lax
from jax.experimental import pallas as pl
from jax.experimental.pallas import tpu as pltpu
from jax.experimental.pallas import tpu_sc as plsc

BRUN = 128
SGRP = 8
CHUNK = 128
NBUF = 10


def _emb_transposed(wordsT, table, seq, batch, dim):
    info = plsc.get_sparse_core_info()
    num_cores = info.num_cores
    n_t = seq // SGRP
    n_tc = batch // BRUN
    mesh = plsc.VectorSubcoreMesh(core_axis_name="c", subcore_axis_name="s")

    @functools.partial(
        pl.kernel,
        out_type=jax.ShapeDtypeStruct((seq, dim // 8, n_tc, 8, BRUN), jnp.float32),
        mesh=mesh,
        scratch_types=[
            pltpu.VMEM((2, SGRP, BRUN), jnp.int32),
            pltpu.VMEM((SGRP, BRUN, dim), jnp.float32),
            pltpu.VMEM((2, dim // 8, 8, BRUN + 1), jnp.float32),
            pltpu.SemaphoreType.DMA((2,)),
            pltpu.SemaphoreType.DMA((SGRP,)),
            pltpu.SemaphoreType.DMA((2,)),
        ],
        compiler_params=pltpu.CompilerParams(
            use_tc_tiling_on_sc=False, needs_layout_passes=False
        ),
    )
    def emb(wordsT_hbm, table_hbm, out_hbm, ib, gb, tb, isem, gsem, wsem):
        w = lax.axis_index("s") * num_cores + lax.axis_index("c")
        b0 = w * BRUN
        iota16 = lax.iota(jnp.int32, 16)
        hi8 = lax.div(iota16, 8)
        lo8 = lax.rem(iota16, 8)

        def ib_copy(t, j):
            return pltpu.make_async_copy(
                wordsT_hbm.at[pl.ds(SGRP * t, SGRP), pl.ds(b0, BRUN)],
                ib.at[j],
                isem.at[j],
            )

        def gather(j_ib, ss):
            return pltpu.make_async_copy(
                table_hbm.at[ib.at[j_ib, ss]], gb.at[ss], gsem.at[ss]
            )

        def write(s, kt):
            return pltpu.make_async_copy(
                tb.at[kt, :, :, pl.ds(0, BRUN)],
                out_hbm.at[s, :, w],
                wsem.at[kt],
            )

        def transpose(ss, kt):
            def body(b, _):
                bvec = jnp.full((16,), b, jnp.int32)
                vs = [gb[ss, b, pl.ds(16 * g, 16)] for g in range(4)]
                for g in range(4):
                    plsc.store_scatter(
                        tb.at[kt], [2 * g + hi8, lo8, bvec], vs[g]
                    )
                return _

            lax.fori_loop(0, BRUN, body, None)

        ib_copy(0, 0).start()
        ib_copy(0, 0).wait()
        for ss in range(SGRP):
            gather(0, ss).start()

        def t_body(t, _):
            j = lax.rem(t, 2)

            @pl.when(t + 1 < n_t)
            def _():
                ib_copy(t + 1, 1 - j).start()

            for ss in range(SGRP):
                s = SGRP * t + ss
                kt = ss % 2
                gather(j, ss).wait()

                @pl.when(s >= 2)
                def _():
                    write(s - 2, kt).wait()

                transpose(ss, kt)
                write(s, kt).start()

                if ss == 0:

                    @pl.when(t + 1 < n_t)
                    def _():
                        ib_copy(t + 1, 1 - j).wait()

                @pl.when(t + 1 < n_t)
                def _():
                    gather(1 - j, ss).start()

            return _

        lax.fori_loop(0, n_t, t_body, None)
        write(seq - 2, 0).wait()
        write(seq - 1, 1).wait()

    return emb(wordsT, table)


def _emb_rowgather(idx3, weights, n_per_w, n_chunks, dim):
    info = plsc.get_sparse_core_info()
    num_cores = info.num_cores
    num_workers = info.num_cores * info.num_subcores
    n_total = num_workers * n_per_w
    mesh = plsc.VectorSubcoreMesh(core_axis_name="c", subcore_axis_name="s")

    @functools.partial(
        pl.kernel,
        out_type=jax.ShapeDtypeStruct((n_total, dim), jnp.float32),
        mesh=mesh,
        scratch_types=[
            pltpu.VMEM((n_chunks, CHUNK), jnp.int32),
            pltpu.VMEM((NBUF, CHUNK, dim), jnp.float32),
            pltpu.SemaphoreType.DMA((NBUF,)),
        ],
        compiler_params=pltpu.CompilerParams(use_tc_tiling_on_sc=False),
    )
    def emb(idx_hbm, table_hbm, out_hbm, idx_v, rows_v, gsem):
        wid = lax.axis_index("s") * num_cores + lax.axis_index("c")
        base = wid * n_per_w
        pltpu.sync_copy(idx_hbm.at[wid], idx_v)

        def gather(c, b):
            return pltpu.make_async_copy(
                table_hbm.at[idx_v.at[c]], rows_v.at[b], gsem.at[b]
            )

        for b in range(min(NBUF, n_chunks)):
            gather(b, b).start()

        def group(g, _):
            for b in range(NBUF):
                c = g * NBUF + b
                gather(c, b).wait()
                pltpu.sync_copy(
                    rows_v.at[b], out_hbm.at[pl.ds(base + c * CHUNK, CHUNK)]
                )
                nxt = c + NBUF

                @pl.when(nxt < n_chunks)
                def _():
                    gather(nxt, b).start()

            return _

        n_full = n_chunks // NBUF
        lax.fori_loop(0, n_full, group, None)
        for b in range(n_chunks % NBUF):
            c = n_full * NBUF + b
            gather(c, b).wait()
            pltpu.sync_copy(
                rows_v.at[b], out_hbm.at[pl.ds(base + c * CHUNK, CHUNK)]
            )

    return emb(idx3, weights)


def kernel(words, weights):
    batch, seq = words.shape
    _, dim = weights.shape
    info = plsc.get_sparse_core_info()
    num_workers = info.num_cores * info.num_subcores

    if (
        dim == 64
        and batch == num_workers * BRUN
        and seq % SGRP == 0
        and seq >= 2 * SGRP
    ):
        wordsT = words.T.astype(jnp.int32)
        out5 = _emb_transposed(wordsT, weights, seq, batch, dim)
        out = out5.transpose(2, 4, 0, 1, 3).reshape(batch, seq, dim)
        return out

    n = batch * seq
    flat = words.reshape(n).astype(jnp.int32)
    tile = num_workers * CHUNK
    n_pad = ((n + tile - 1) // tile) * tile
    if n_pad != n:
        flat = jnp.concatenate([flat, jnp.zeros(n_pad - n, jnp.int32)])
    n_per_w = n_pad // num_workers
    n_chunks = n_per_w // CHUNK

    idx3 = flat.reshape(num_workers, n_chunks, CHUNK)
    out = _emb_rowgather(idx3, weights, n_per_w, n_chunks, dim)
    return out[:n].reshape(batch, seq, dim)
